# baseline (speedup 1.0000x reference)
import functools

import jax
import jax.numpy as jnp
from jax.experimental import pallas as pl
from jax.experimental.pallas import tpu as pltpu

TAU = 0.07
MARGIN = 0.3


def _loss_body(C, BB, logits_ref, labels_ref, tlt_ref, tft_ref, proto_ref, out_ref):
    f32 = jnp.float32
    x = logits_ref[...]
    y = labels_ref[...]

    bce = jnp.maximum(x, 0.0) - x * y + jnp.log(1.0 + jnp.exp(-jnp.abs(x)))
    bce_s = jnp.sum(bce, axis=1, keepdims=True)

    y_t = y.T
    msim = jnp.max(tlt_ref[...], axis=2)
    sig = 1.0 / (1.0 + jnp.exp(-msim))
    rej = jnp.maximum(sig - MARGIN, 0.0)
    neg_t = jnp.where(y_t > 0.0, 0.0, 1.0).astype(f32)
    rej_s = jnp.sum(rej * neg_t, axis=0, keepdims=True)

    pt = proto_ref[...]
    pn = pt / jnp.maximum(
        jnp.sqrt(jnp.sum(pt * pt, axis=1, keepdims=True)), 1e-12)
    con_s = jnp.zeros((BB, 1), dtype=f32)
    for c in range(C):
        f = tft_ref[c]
        nrm = jnp.sqrt(jnp.sum(f * f, axis=1, keepdims=True))
        fn = f / jnp.maximum(nrm, 1e-12)
        sim = jax.lax.dot_general(
            fn, pn, (((1,), (1,)), ((), ())),
            preferred_element_type=f32) * (1.0 / TAU)
        m = jnp.max(sim, axis=1, keepdims=True)
        lse = m + jnp.log(jnp.sum(jnp.exp(sim - m), axis=1, keepdims=True))
        diag = sim[:, c:c + 1]
        psc = lse - diag
        con_s = con_s + psc * y[:, c:c + 1]
    total = (bce_s + con_s).T + rej_s
    out_ref[...] = total.reshape(BB)


def kernel(logits, total_cls_logits, total_cls_feature, labels, prototypes):
    B, C = logits.shape
    _, _, L = total_cls_logits.shape
    _, _, D = total_cls_feature.shape
    BB = 256
    NB = B // BB

    grid_spec = pl.GridSpec(
        grid=(NB,),
        in_specs=[
            pl.BlockSpec((BB, C), lambda i: (i, 0)),
            pl.BlockSpec((BB, C), lambda i: (i, 0)),
            pl.BlockSpec((C, BB, L), lambda i: (0, i, 0)),
            pl.BlockSpec((C, BB, D), lambda i: (0, i, 0)),
            pl.BlockSpec((C, D), lambda i: (0, 0)),
        ],
        out_specs=pl.BlockSpec((BB,), lambda i: (i,)),
    )
    out = pl.pallas_call(
        functools.partial(_loss_body, C, BB),
        grid_spec=grid_spec,
        out_shape=jax.ShapeDtypeStruct((B,), jnp.float32),
        compiler_params=pltpu.CompilerParams(
            dimension_semantics=("arbitrary",),
        ),
    )(logits, labels, total_cls_logits, total_cls_feature, prototypes)
    return out


# speedup vs baseline: 2.2490x; 2.2490x over previous
import functools

import jax
import jax.numpy as jnp
from jax.experimental import pallas as pl
from jax.experimental.pallas import tpu as pltpu

TAU = 0.07
MARGIN = 0.3


def _loss_body(C, BB, xt_ref, yt_ref, tlt_ref, tft_ref, proto_ref, out_ref):
    f32 = jnp.float32
    x = xt_ref[...]
    y = yt_ref[...]

    bce = jnp.maximum(x, 0.0) - x * y + jnp.log(1.0 + jnp.exp(-jnp.abs(x)))
    bce_s = jnp.sum(bce, axis=0, keepdims=True)

    msim = jnp.max(tlt_ref[...], axis=2)
    sig = 1.0 / (1.0 + jnp.exp(-msim))
    rej = jnp.maximum(sig - MARGIN, 0.0)
    rej_s = jnp.sum(rej * (1.0 - y), axis=0, keepdims=True)

    tft = tft_ref[...]
    inv_nrm = 1.0 / jnp.maximum(
        jnp.sqrt(jnp.sum(tft * tft, axis=2)), 1e-12)
    pt = proto_ref[...]
    pn = pt / jnp.maximum(
        jnp.sqrt(jnp.sum(pt * pt, axis=1, keepdims=True)), 1e-12)
    acc = bce_s + rej_s
    for c in range(C):
        f = tft_ref[c]
        s = jax.lax.dot_general(
            pn, f, (((1,), (1,)), ((), ())),
            preferred_element_type=f32)
        s = s * (inv_nrm[c:c + 1, :] * (1.0 / TAU))
        m = jnp.max(s, axis=0, keepdims=True)
        lse = m + jnp.log(jnp.sum(jnp.exp(s - m), axis=0, keepdims=True))
        psc = lse - s[c:c + 1, :]
        acc = acc + psc * y[c:c + 1, :]
    out_ref[...] = acc.reshape(BB)


def kernel(logits, total_cls_logits, total_cls_feature, labels, prototypes):
    B, C = logits.shape
    _, _, L = total_cls_logits.shape
    _, _, D = total_cls_feature.shape
    BB = 256
    NB = B // BB

    xt = logits.T
    yt = labels.T

    grid_spec = pl.GridSpec(
        grid=(NB,),
        in_specs=[
            pl.BlockSpec((C, BB), lambda i: (0, i)),
            pl.BlockSpec((C, BB), lambda i: (0, i)),
            pl.BlockSpec((C, BB, L), lambda i: (0, i, 0)),
            pl.BlockSpec((C, BB, D), lambda i: (0, i, 0)),
            pl.BlockSpec((C, D), lambda i: (0, 0)),
        ],
        out_specs=pl.BlockSpec((BB,), lambda i: (i,)),
    )
    out = pl.pallas_call(
        functools.partial(_loss_body, C, BB),
        grid_spec=grid_spec,
        out_shape=jax.ShapeDtypeStruct((B,), jnp.float32),
        compiler_params=pltpu.CompilerParams(
            dimension_semantics=("arbitrary",),
        ),
    )(xt, yt, total_cls_logits, total_cls_feature, prototypes)
    return out


# speedup vs baseline: 2.9601x; 1.3162x over previous
import functools

import jax
import jax.numpy as jnp
from jax.experimental import pallas as pl
from jax.experimental.pallas import tpu as pltpu

TAU = 0.07
MARGIN = 0.3


def _loss_body(C, BB, D, xt_ref, yt_ref, tlt_ref, tft_ref, proto_ref, out_ref):
    f32 = jnp.float32
    x = xt_ref[...]
    y = yt_ref[...]

    bce = jnp.maximum(x, 0.0) - x * y + jnp.log(1.0 + jnp.exp(-jnp.abs(x)))
    bce_s = jnp.sum(bce, axis=0, keepdims=True)

    msim = jnp.max(tlt_ref[...], axis=2)
    sig = 1.0 / (1.0 + jnp.exp(-msim))
    rej = jnp.maximum(sig - MARGIN, 0.0)
    rej_s = jnp.sum(rej * (1.0 - y), axis=0, keepdims=True)

    f2 = tft_ref[...].reshape(C * BB, D)
    pt = proto_ref[...]
    pn = pt / jnp.maximum(
        jnp.sqrt(jnp.sum(pt * pt, axis=1, keepdims=True)), 1e-12)
    ones_d = jnp.ones((1, D), dtype=f32)
    nrm2 = jax.lax.dot_general(
        ones_d, f2 * f2, (((1,), (1,)), ((), ())),
        preferred_element_type=f32)
    inv = (1.0 / TAU) / jnp.maximum(jnp.sqrt(nrm2), 1e-12)
    s = jax.lax.dot_general(
        pn, f2, (((1,), (1,)), ((), ())),
        preferred_element_type=f32)
    s = s * inv
    m = jnp.max(s, axis=0, keepdims=True)
    lse = m + jnp.log(jnp.sum(jnp.exp(s - m), axis=0, keepdims=True))
    acc = bce_s + rej_s
    for c in range(C):
        psc = lse[:, c * BB:(c + 1) * BB] - s[c:c + 1, c * BB:(c + 1) * BB]
        acc = acc + psc * y[c:c + 1, :]
    out_ref[...] = acc.reshape(BB)


def kernel(logits, total_cls_logits, total_cls_feature, labels, prototypes):
    B, C = logits.shape
    _, _, L = total_cls_logits.shape
    _, _, D = total_cls_feature.shape
    BB = 256
    NB = B // BB

    xt = logits.T
    yt = labels.T

    grid_spec = pl.GridSpec(
        grid=(NB,),
        in_specs=[
            pl.BlockSpec((C, BB), lambda i: (0, i)),
            pl.BlockSpec((C, BB), lambda i: (0, i)),
            pl.BlockSpec((C, BB, L), lambda i: (0, i, 0)),
            pl.BlockSpec((C, BB, D), lambda i: (0, i, 0)),
            pl.BlockSpec((C, D), lambda i: (0, 0)),
        ],
        out_specs=pl.BlockSpec((BB,), lambda i: (i,)),
    )
    out = pl.pallas_call(
        functools.partial(_loss_body, C, BB, D),
        grid_spec=grid_spec,
        out_shape=jax.ShapeDtypeStruct((B,), jnp.float32),
        compiler_params=pltpu.CompilerParams(
            dimension_semantics=("arbitrary",),
        ),
    )(xt, yt, total_cls_logits, total_cls_feature, prototypes)
    return out


# speedup vs baseline: 2.9872x; 1.0091x over previous
import functools

import jax
import jax.numpy as jnp
from jax.experimental import pallas as pl
from jax.experimental.pallas import tpu as pltpu

TAU = 0.07
MARGIN = 0.3


def _loss_body(C, BB, D, xt_ref, yt_ref, tlt_ref, tft_ref, proto_ref, out_ref):
    f32 = jnp.float32
    x = xt_ref[...]
    y = yt_ref[...]

    bce = jnp.maximum(x, 0.0) - x * y + jnp.log(1.0 + jnp.exp(-jnp.abs(x)))
    bce_s = jnp.sum(bce, axis=0, keepdims=True)

    msim = jnp.max(tlt_ref[...], axis=2)
    sig = 1.0 / (1.0 + jnp.exp(-msim))
    rej = jnp.maximum(sig - MARGIN, 0.0)
    rej_s = jnp.sum(rej * (1.0 - y), axis=0, keepdims=True)

    f2 = tft_ref[...].reshape(C * BB, D)
    pt = proto_ref[...]
    pn = pt / jnp.maximum(
        jnp.sqrt(jnp.sum(pt * pt, axis=1, keepdims=True)), 1e-12)
    ones_d = jnp.ones((1, D), dtype=f32)
    nrm2 = jax.lax.dot_general(
        ones_d, f2 * f2, (((1,), (1,)), ((), ())),
        preferred_element_type=f32)
    inv = (1.0 / TAU) / jnp.maximum(jnp.sqrt(nrm2), 1e-12)
    s = jax.lax.dot_general(
        pn, f2, (((1,), (1,)), ((), ())),
        preferred_element_type=f32)
    s = s * inv
    m = jnp.max(s, axis=0, keepdims=True)
    lse = m + jnp.log(jnp.sum(jnp.exp(s - m), axis=0, keepdims=True))
    acc = bce_s + rej_s
    for c in range(C):
        psc = lse[:, c * BB:(c + 1) * BB] - s[c:c + 1, c * BB:(c + 1) * BB]
        acc = acc + psc * y[c:c + 1, :]
    out_ref[...] = acc.reshape(BB)


def kernel(logits, total_cls_logits, total_cls_feature, labels, prototypes):
    B, C = logits.shape
    _, _, L = total_cls_logits.shape
    _, _, D = total_cls_feature.shape
    BB = 512
    NB = B // BB

    xt = logits.T
    yt = labels.T

    grid_spec = pl.GridSpec(
        grid=(NB,),
        in_specs=[
            pl.BlockSpec((C, BB), lambda i: (0, i)),
            pl.BlockSpec((C, BB), lambda i: (0, i)),
            pl.BlockSpec((C, BB, L), lambda i: (0, i, 0)),
            pl.BlockSpec((C, BB, D), lambda i: (0, i, 0)),
            pl.BlockSpec((C, D), lambda i: (0, 0)),
        ],
        out_specs=pl.BlockSpec((BB,), lambda i: (i,)),
    )
    out = pl.pallas_call(
        functools.partial(_loss_body, C, BB, D),
        grid_spec=grid_spec,
        out_shape=jax.ShapeDtypeStruct((B,), jnp.float32),
        compiler_params=pltpu.CompilerParams(
            dimension_semantics=("arbitrary",),
        ),
    )(xt, yt, total_cls_logits, total_cls_feature, prototypes)
    return out


# speedup vs baseline: 3.7511x; 1.2557x over previous
import functools

import jax
import jax.numpy as jnp
from jax import lax
from jax.experimental import pallas as pl
from jax.experimental.pallas import tpu as pltpu
from jax.experimental.pallas import tpu_sc as plsc

TAU = 0.07
MARGIN = 0.3


def _rejection_sc(tbl_flat, labt_flat, C, B, L):
    info = plsc.get_sparse_core_info()
    NW = info.num_cores * info.num_subcores
    SB = B // NW
    GR = 64
    IPC = SB // GR
    NCH = C * IPC
    CH = GR * L
    mesh = plsc.VectorSubcoreMesh(core_axis_name="c", subcore_axis_name="s")

    @functools.partial(
        pl.kernel, mesh=mesh,
        out_type=jax.ShapeDtypeStruct((B,), jnp.float32),
        scratch_types=[
            pltpu.VMEM((C * SB,), jnp.float32),
            pltpu.VMEM((CH,), jnp.float32),
            pltpu.VMEM((CH,), jnp.float32),
            pltpu.VMEM((SB,), jnp.float32),
            pltpu.SemaphoreType.DMA,
            pltpu.SemaphoreType.DMA,
        ],
    )
    def sc_kernel(tbl_hbm, labt_hbm, out_hbm, lab_v, rows_a, rows_b, acc_v,
                  sem_a, sem_b):
        wid = lax.axis_index("s") * info.num_cores + lax.axis_index("c")
        b0 = wid * SB
        iota = lax.iota(jnp.int32, 16)

        for c in range(C):
            pltpu.sync_copy(labt_hbm.at[pl.ds(c * B + b0, SB)],
                            lab_v.at[pl.ds(c * SB, SB)])

        def zbody(i, carry):
            acc_v[pl.ds(i * 16, 16)] = jnp.zeros((16,), jnp.float32)
            return carry
        lax.fori_loop(0, SB // 16, zbody, 0)

        def chunk_off(t):
            c = t // IPC
            return (c * B + b0 + (t - c * IPC) * GR) * L

        def start(t, buf, sem):
            pltpu.make_async_copy(
                tbl_hbm.at[pl.ds(chunk_off(t), CH)], buf, sem).start()

        def wait(t, buf, sem):
            pltpu.make_async_copy(
                tbl_hbm.at[pl.ds(chunk_off(t), CH)], buf, sem).wait()

        def process(t, buf):
            c = t // IPC
            bl0 = (t - c * IPC) * GR
            for g in range(GR // 16):
                acc16 = jnp.zeros((16,), jnp.float32)
                for r in range(16):
                    row = (g * 16 + r) * L
                    m = buf[pl.ds(row, 16)]
                    for w in range(1, L // 16):
                        m = jnp.maximum(m, buf[pl.ds(row + w * 16, 16)])
                    for st in (8, 4, 2, 1):
                        sh = m.at[iota ^ st].get(mode="promise_in_bounds")
                        m = jnp.maximum(m, sh)
                    acc16 = jnp.where(iota == r, m, acc16)
                sig = 1.0 / (1.0 + jnp.exp(-acc16))
                rj = jnp.maximum(sig - MARGIN, 0.0)
                w16 = lab_v[pl.ds(c * SB + bl0 + g * 16, 16)]
                rj = rj * (1.0 - w16)
                sl = pl.ds(bl0 + g * 16, 16)
                acc_v[sl] = acc_v[sl] + rj

        start(0, rows_a, sem_a)

        def body(u, carry):
            ta = 2 * u
            tb = 2 * u + 1
            start(tb, rows_b, sem_b)
            wait(ta, rows_a, sem_a)
            process(ta, rows_a)

            @pl.when(ta + 2 < NCH)
            def _():
                start(ta + 2, rows_a, sem_a)
            wait(tb, rows_b, sem_b)
            process(tb, rows_b)
            return carry
        lax.fori_loop(0, NCH // 2, body, 0)

        pltpu.sync_copy(acc_v, out_hbm.at[pl.ds(b0, SB)])

    return sc_kernel(tbl_flat, labt_flat)


def _bce_con_body(C, BB, D, xt_ref, yt_ref, tft_ref, proto_ref, out_ref):
    f32 = jnp.float32
    x = xt_ref[...]
    y = yt_ref[...]

    bce = jnp.maximum(x, 0.0) - x * y + jnp.log(1.0 + jnp.exp(-jnp.abs(x)))
    bce_s = jnp.sum(bce, axis=0, keepdims=True)

    f2 = tft_ref[...].reshape(C * BB, D)
    pt = proto_ref[...]
    pn = pt / jnp.maximum(
        jnp.sqrt(jnp.sum(pt * pt, axis=1, keepdims=True)), 1e-12)
    ones_d = jnp.ones((1, D), dtype=f32)
    nrm2 = jax.lax.dot_general(
        ones_d, f2 * f2, (((1,), (1,)), ((), ())),
        preferred_element_type=f32)
    inv = (1.0 / TAU) / jnp.maximum(jnp.sqrt(nrm2), 1e-12)
    s = jax.lax.dot_general(
        pn, f2, (((1,), (1,)), ((), ())),
        preferred_element_type=f32)
    s = s * inv
    m = jnp.max(s, axis=0, keepdims=True)
    lse = m + jnp.log(jnp.sum(jnp.exp(s - m), axis=0, keepdims=True))
    acc = bce_s
    for c in range(C):
        psc = lse[:, c * BB:(c + 1) * BB] - s[c:c + 1, c * BB:(c + 1) * BB]
        acc = acc + psc * y[c:c + 1, :]
    out_ref[...] = acc.reshape(BB)


def _bce_con_tc(logits, total_cls_feature, labels, prototypes):
    B, C = logits.shape
    _, _, D = total_cls_feature.shape
    BB = 512
    NB = B // BB
    xt = logits.T
    yt = labels.T
    grid_spec = pl.GridSpec(
        grid=(NB,),
        in_specs=[
            pl.BlockSpec((C, BB), lambda i: (0, i)),
            pl.BlockSpec((C, BB), lambda i: (0, i)),
            pl.BlockSpec((C, BB, D), lambda i: (0, i, 0)),
            pl.BlockSpec((C, D), lambda i: (0, 0)),
        ],
        out_specs=pl.BlockSpec((BB,), lambda i: (i,)),
    )
    return pl.pallas_call(
        functools.partial(_bce_con_body, C, BB, D),
        grid_spec=grid_spec,
        out_shape=jax.ShapeDtypeStruct((B,), jnp.float32),
        compiler_params=pltpu.CompilerParams(
            dimension_semantics=("arbitrary",),
        ),
    )(xt, yt, total_cls_feature, prototypes)


def kernel(logits, total_cls_logits, total_cls_feature, labels, prototypes):
    C, B, L = total_cls_logits.shape
    rej = _rejection_sc(total_cls_logits.reshape(C * B * L),
                        labels.T.reshape(C * B), C, B, L)
    rest = _bce_con_tc(logits, total_cls_feature, labels, prototypes)
    return rest + rej
